# Initial kernel scaffold; baseline (speedup 1.0000x reference)
import functools

import jax
import jax.numpy as jnp
from jax import lax
from jax.experimental import pallas as pl
from jax.experimental.pallas import tpu as pltpu
from jax.experimental.pallas import tpu_sc as plsc

N = 10000
E = 320000
D = 128

NC = 2
NS = 16
NW = NC * NS
EPW = E // NW
CHUNK = 80
NCHUNK = EPW // CHUNK
ROWS_PER_TILE = N // NS


def _matmul_kernel(x_ref, w_ref, b_ref, o_ref):
    o_ref[...] = (
        jnp.dot(x_ref[...], w_ref[...], preferred_element_type=jnp.float32)
        + b_ref[...]
    )


def _add_kernel(a_ref, b_ref, o_ref):
    o_ref[...] = a_ref[...] + b_ref[...]


def _spmm_body(h_hbm, src_hbm, dst_hbm, val_hbm, zero_hbm, out_hbm,
               src_v, dst_v, val_v, rows_v, acc_shared, sem):
    cid = lax.axis_index("c")
    sid = lax.axis_index("s")
    wid = sid * NC + cid
    base = wid * EPW

    pltpu.sync_copy(
        zero_hbm.at[pl.ds(sid * ROWS_PER_TILE, ROWS_PER_TILE)],
        acc_shared.at[pl.ds(sid * ROWS_PER_TILE, ROWS_PER_TILE)],
    )
    plsc.subcore_barrier()

    def chunk_body(ci, carry):
        off = base + ci * CHUNK
        pltpu.sync_copy(src_hbm.at[pl.ds(off, CHUNK)], src_v)
        pltpu.sync_copy(dst_hbm.at[pl.ds(off, CHUNK)], dst_v)
        pltpu.sync_copy(val_hbm.at[pl.ds(off, CHUNK)], val_v)
        pltpu.async_copy(h_hbm.at[src_v], rows_v, sem).wait()

        def edge_body(e, carry2):
            v = val_v[e]
            for j in range(D // 16):
                sl = pl.ds(j * 16, 16)
                rows_v[e, sl] = rows_v[e, sl] * v
            return carry2

        lax.fori_loop(0, CHUNK, edge_body, 0, unroll=2)
        pltpu.sync_copy(rows_v, acc_shared.at[dst_v], add=True)
        return carry

    lax.fori_loop(0, NCHUNK, chunk_body, 0)
    plsc.subcore_barrier()

    pltpu.sync_copy(
        acc_shared.at[pl.ds(sid * ROWS_PER_TILE, ROWS_PER_TILE)],
        out_hbm.at[cid, pl.ds(sid * ROWS_PER_TILE, ROWS_PER_TILE)],
    )


@jax.jit
def _spmm(h, src, dst, val, zero):
    mesh = plsc.VectorSubcoreMesh(core_axis_name="c", subcore_axis_name="s")
    f = pl.kernel(
        _spmm_body,
        out_type=jax.ShapeDtypeStruct((NC, N, D), jnp.float32),
        mesh=mesh,
        scratch_types=[
            pltpu.VMEM((CHUNK,), jnp.int32),
            pltpu.VMEM((CHUNK,), jnp.int32),
            pltpu.VMEM((CHUNK,), jnp.float32),
            pltpu.VMEM((CHUNK, D), jnp.float32),
            pltpu.VMEM_SHARED((N, D), jnp.float32),
            pltpu.SemaphoreType.DMA,
        ],
    )
    return f(h, src, dst, val, zero)


def kernel(x, adj_indices, adj_values, W, b):
    h = pl.pallas_call(
        _matmul_kernel,
        grid=(10,),
        in_specs=[
            pl.BlockSpec((N // 10, D), lambda i: (i, 0)),
            pl.BlockSpec((D, D), lambda i: (0, 0)),
            pl.BlockSpec((1, D), lambda i: (0, 0)),
        ],
        out_specs=pl.BlockSpec((N // 10, D), lambda i: (i, 0)),
        out_shape=jax.ShapeDtypeStruct((N, D), jnp.float32),
    )(x, W, b.reshape(1, D))

    dst = adj_indices[0]
    src = adj_indices[1]
    zero = jnp.zeros((N, D), jnp.float32)
    partials = _spmm(h, src, dst, adj_values, zero)

    out = pl.pallas_call(
        _add_kernel,
        grid=(10,),
        in_specs=[
            pl.BlockSpec((N // 10, D), lambda i: (i, 0)),
            pl.BlockSpec((N // 10, D), lambda i: (i, 0)),
        ],
        out_specs=pl.BlockSpec((N // 10, D), lambda i: (i, 0)),
        out_shape=jax.ShapeDtypeStruct((N, D), jnp.float32),
    )(partials[0], partials[1])
    return out


# speedup vs baseline: 4.2935x; 4.2935x over previous
import functools

import jax
import jax.numpy as jnp
from jax import lax
from jax.experimental import pallas as pl
from jax.experimental.pallas import tpu as pltpu
from jax.experimental.pallas import tpu_sc as plsc

N = 10000
E = 320000
D = 128

NC = 2
NS = 16
NW = NC * NS
EPW = E // NW
CHUNK = 80
NCHUNK = EPW // CHUNK
NP = 10240
ROWS_PER_TILE = NP // NS


def _matmul_kernel(x_ref, w_ref, b_ref, o_ref):
    o_ref[...] = (
        jnp.dot(x_ref[...], w_ref[...], preferred_element_type=jnp.float32)
        + b_ref[...]
    )


def _add_kernel(a_ref, b_ref, o_ref):
    o_ref[...] = a_ref[...] + b_ref[...]


def _spmm_body(h_hbm, src_hbm, dst_hbm, val_hbm, zero_hbm, out_hbm,
               src_v, dst_v, val_v, rows_v, acc_shared, sem):
    cid = lax.axis_index("c")
    sid = lax.axis_index("s")
    wid = sid * NC + cid
    base = wid * EPW

    pltpu.sync_copy(
        zero_hbm.at[pl.ds(sid * ROWS_PER_TILE, ROWS_PER_TILE)],
        acc_shared.at[pl.ds(sid * ROWS_PER_TILE, ROWS_PER_TILE)],
    )
    plsc.subcore_barrier()

    def chunk_body(ci, carry):
        off = base + ci * CHUNK
        pltpu.sync_copy(src_hbm.at[pl.ds(off, CHUNK)], src_v)
        pltpu.sync_copy(dst_hbm.at[pl.ds(off, CHUNK)], dst_v)
        pltpu.sync_copy(val_hbm.at[pl.ds(off, CHUNK)], val_v)
        pltpu.async_copy(h_hbm.at[src_v], rows_v, sem).wait()

        def group_body(g, carry2):
            gbase = g * 16
            vv = val_v[pl.ds(gbase, 16)]
            for e in range(16):
                v = vv[e]
                for j in range(D // 16):
                    sl = pl.ds(j * 16, 16)
                    rows_v[gbase + e, sl] = rows_v[gbase + e, sl] * v
            return carry2

        lax.fori_loop(0, CHUNK // 16, group_body, 0)
        pltpu.sync_copy(rows_v, acc_shared.at[dst_v], add=True)
        return carry

    lax.fori_loop(0, NCHUNK, chunk_body, 0)
    plsc.subcore_barrier()

    pltpu.sync_copy(
        acc_shared.at[pl.ds(sid * ROWS_PER_TILE, ROWS_PER_TILE)],
        out_hbm.at[cid, pl.ds(sid * ROWS_PER_TILE, ROWS_PER_TILE)],
    )


@jax.jit
def _spmm(h, src, dst, val, zero):
    mesh = plsc.VectorSubcoreMesh(core_axis_name="c", subcore_axis_name="s")
    f = pl.kernel(
        _spmm_body,
        out_type=jax.ShapeDtypeStruct((NC, NP, D), jnp.float32),
        mesh=mesh,
        scratch_types=[
            pltpu.VMEM((CHUNK,), jnp.int32),
            pltpu.VMEM((CHUNK,), jnp.int32),
            pltpu.VMEM((CHUNK,), jnp.float32),
            pltpu.VMEM((CHUNK, D), jnp.float32),
            pltpu.VMEM_SHARED((NP, D), jnp.float32),
            pltpu.SemaphoreType.DMA,
        ],
    )
    return f(h, src, dst, val, zero)


def kernel(x, adj_indices, adj_values, W, b):
    h = pl.pallas_call(
        _matmul_kernel,
        grid=(10,),
        in_specs=[
            pl.BlockSpec((N // 10, D), lambda i: (i, 0)),
            pl.BlockSpec((D, D), lambda i: (0, 0)),
            pl.BlockSpec((1, D), lambda i: (0, 0)),
        ],
        out_specs=pl.BlockSpec((N // 10, D), lambda i: (i, 0)),
        out_shape=jax.ShapeDtypeStruct((N, D), jnp.float32),
    )(x, W, b.reshape(1, D))

    dst = adj_indices[0]
    src = adj_indices[1]
    zero = jnp.zeros((NP, D), jnp.float32)
    partials = _spmm(h, src, dst, adj_values, zero)

    out = pl.pallas_call(
        _add_kernel,
        grid=(10,),
        in_specs=[
            pl.BlockSpec((NP // 10, D), lambda i: (i, 0)),
            pl.BlockSpec((NP // 10, D), lambda i: (i, 0)),
        ],
        out_specs=pl.BlockSpec((NP // 10, D), lambda i: (i, 0)),
        out_shape=jax.ShapeDtypeStruct((NP, D), jnp.float32),
    )(partials[0], partials[1])
    return out[:N]


# speedup vs baseline: 8.0100x; 1.8656x over previous
import jax
import jax.numpy as jnp
from jax import lax
from jax.experimental import pallas as pl
from jax.experimental.pallas import tpu as pltpu
from jax.experimental.pallas import tpu_sc as plsc

N = 10000
E = 320000
D = 128

NC = 2
NS = 16
NW = NC * NS
EPW = E // NW
CHUNK = 80
NCHUNK = EPW // CHUNK
NP = 10240
ROWS_PER_TILE = NP // NS

NRB = 4
NIB = 8


def _matmul_kernel(x_ref, w_ref, b_ref, o_ref):
    o_ref[...] = (
        jnp.dot(x_ref[...], w_ref[...], preferred_element_type=jnp.float32)
        + b_ref[...]
    )


def _add_kernel(a_ref, b_ref, o_ref):
    o_ref[...] = a_ref[...] + b_ref[...]


def _spmm_body(h_hbm, src_hbm, dst_hbm, val_hbm, zero_hbm, out_hbm,
               srcb, dstb, valb, rows, acc_shared, srs, dss, vls, gs):
    cid = lax.axis_index("c")
    sid = lax.axis_index("s")
    wid = sid * NC + cid

    pltpu.sync_copy(
        zero_hbm.at[pl.ds(sid * ROWS_PER_TILE, ROWS_PER_TILE)],
        acc_shared.at[pl.ds(sid * ROWS_PER_TILE, ROWS_PER_TILE)],
    )
    plsc.subcore_barrier()

    ebase = wid * EPW

    def start_idx(c, m):
        off = ebase + c * CHUNK
        pltpu.async_copy(src_hbm.at[pl.ds(off, CHUNK)], srcb[m], srs[m])
        pltpu.async_copy(dst_hbm.at[pl.ds(off, CHUNK)], dstb[m], dss[m])
        pltpu.async_copy(val_hbm.at[pl.ds(off, CHUNK)], valb[m], vls[m])

    def wait_idx(c, m):
        off = ebase + c * CHUNK
        pltpu.make_async_copy(src_hbm.at[pl.ds(off, CHUNK)], srcb[m], srs[m]).wait()
        pltpu.make_async_copy(dst_hbm.at[pl.ds(off, CHUNK)], dstb[m], dss[m]).wait()
        pltpu.make_async_copy(val_hbm.at[pl.ds(off, CHUNK)], valb[m], vls[m]).wait()

    def start_gather(m):
        pltpu.async_copy(h_hbm.at[srcb[m]], rows[m], gs[m])

    def wait_gather(m):
        pltpu.make_async_copy(h_hbm.at[srcb[m]], rows[m], gs[m]).wait()

    def scatter_sync(m):
        pltpu.sync_copy(rows[m], acc_shared.at[dstb[m]], add=True)

    def scale(m):
        buf = rows[m]
        vref = valb[m]

        def group_body(g, carry):
            gbase = g * 16
            vv = vref[pl.ds(gbase, 16)]
            for e in range(16):
                v = vv[e]
                for j in range(D // 16):
                    sl = pl.ds(j * 16, 16)
                    buf[gbase + e, sl] = buf[gbase + e, sl] * v
            return carry

        lax.fori_loop(0, CHUNK // 16, group_body, 0)

    start_idx(0, 0)
    start_idx(1, 1)
    wait_idx(0, 0)
    start_gather(0)

    def pair_body(p, carry):
        c0 = 2 * p
        c1 = c0 + 1
        wait_idx(c1, 1)
        start_gather(1)
        wait_gather(0)
        scale(0)
        scatter_sync(0)
        start_idx(c0 + 2, 0)
        wait_gather(1)
        scale(1)
        scatter_sync(1)
        wait_idx(c0 + 2, 0)
        start_gather(0)
        start_idx(jnp.minimum(c1 + 2, NCHUNK - 1), 1)
        return carry

    lax.fori_loop(0, (NCHUNK - 1) // 2, pair_body, 0)

    wait_gather(0)
    scale(0)
    scatter_sync(0)
    wait_idx(NCHUNK - 1, 1)

    plsc.subcore_barrier()

    pltpu.sync_copy(
        acc_shared.at[pl.ds(sid * ROWS_PER_TILE, ROWS_PER_TILE)],
        out_hbm.at[cid, pl.ds(sid * ROWS_PER_TILE, ROWS_PER_TILE)],
    )


@jax.jit
def _spmm(h, src1, dst1, val1, zero):
    mesh = plsc.VectorSubcoreMesh(core_axis_name="c", subcore_axis_name="s")
    f = pl.kernel(
        _spmm_body,
        out_type=jax.ShapeDtypeStruct((NC, NP, D), jnp.float32),
        mesh=mesh,
        scratch_types=[
            [pltpu.VMEM((CHUNK,), jnp.int32) for _ in range(2)],
            [pltpu.VMEM((CHUNK,), jnp.int32) for _ in range(2)],
            [pltpu.VMEM((CHUNK,), jnp.float32) for _ in range(2)],
            [pltpu.VMEM((CHUNK, D), jnp.float32) for _ in range(2)],
            pltpu.VMEM_SHARED((NP, D), jnp.float32),
            [pltpu.SemaphoreType.DMA for _ in range(2)],
            [pltpu.SemaphoreType.DMA for _ in range(2)],
            [pltpu.SemaphoreType.DMA for _ in range(2)],
            [pltpu.SemaphoreType.DMA for _ in range(2)],
        ],
    )
    return f(h, src1, dst1, val1, zero)


def kernel(x, adj_indices, adj_values, W, b):
    h = pl.pallas_call(
        _matmul_kernel,
        grid=(10,),
        in_specs=[
            pl.BlockSpec((N // 10, D), lambda i: (i, 0)),
            pl.BlockSpec((D, D), lambda i: (0, 0)),
            pl.BlockSpec((1, D), lambda i: (0, 0)),
        ],
        out_specs=pl.BlockSpec((N // 10, D), lambda i: (i, 0)),
        out_shape=jax.ShapeDtypeStruct((N, D), jnp.float32),
    )(x, W, b.reshape(1, D))

    dst1 = adj_indices[0]
    src1 = adj_indices[1]
    zero = jnp.zeros((NP, D), jnp.float32)
    partials = _spmm(h, src1, dst1, adj_values, zero)

    out = pl.pallas_call(
        _add_kernel,
        grid=(10,),
        in_specs=[
            pl.BlockSpec((NP // 10, D), lambda i: (i, 0)),
            pl.BlockSpec((NP // 10, D), lambda i: (i, 0)),
        ],
        out_specs=pl.BlockSpec((NP // 10, D), lambda i: (i, 0)),
        out_shape=jax.ShapeDtypeStruct((NP, D), jnp.float32),
    )(partials[0], partials[1])
    return out[:N]


# speedup vs baseline: 8.3276x; 1.0396x over previous
import jax
import jax.numpy as jnp
from jax import lax
from jax.experimental import pallas as pl
from jax.experimental.pallas import tpu as pltpu
from jax.experimental.pallas import tpu_sc as plsc

N = 10000
E = 320000
D = 128

NC = 2
NS = 16
NW = NC * NS
EPW = E // NW
CHUNK = 80
NCHUNK = EPW // CHUNK
NP = 10240
ROWS_PER_TILE = NP // NS

NRB = 4
NIB = 8


def _matmul_kernel(x_ref, w_ref, b_ref, o_ref):
    o_ref[...] = (
        jnp.dot(x_ref[...], w_ref[...], preferred_element_type=jnp.float32)
        + b_ref[...]
    )


def _add_kernel(a_ref, b_ref, o_ref):
    o_ref[...] = a_ref[...] + b_ref[...]


def _spmm_body(h_hbm, src_hbm, dst_hbm, val_hbm, out_hbm,
               srcb, dstb, valb, rows, zbuf, acc_shared, srs, dss, vls, gs):
    cid = lax.axis_index("c")
    sid = lax.axis_index("s")
    wid = sid * NC + cid

    zv = jnp.zeros((16,), jnp.float32)

    def zrow(r, carry):
        for j in range(D // 16):
            zbuf[r, pl.ds(j * 16, 16)] = zv
        return carry

    lax.fori_loop(0, 128, zrow, 0)
    for t in range(ROWS_PER_TILE // 128):
        pltpu.sync_copy(
            zbuf,
            acc_shared.at[pl.ds(sid * ROWS_PER_TILE + t * 128, 128)],
        )
    plsc.subcore_barrier()

    ebase = wid * EPW

    def start_idx(c, m):
        off = ebase + c * CHUNK
        pltpu.async_copy(src_hbm.at[pl.ds(off, CHUNK)], srcb[m], srs[m])
        pltpu.async_copy(dst_hbm.at[pl.ds(off, CHUNK)], dstb[m], dss[m])
        pltpu.async_copy(val_hbm.at[pl.ds(off, CHUNK)], valb[m], vls[m])

    def wait_idx(c, m):
        off = ebase + c * CHUNK
        pltpu.make_async_copy(src_hbm.at[pl.ds(off, CHUNK)], srcb[m], srs[m]).wait()
        pltpu.make_async_copy(dst_hbm.at[pl.ds(off, CHUNK)], dstb[m], dss[m]).wait()
        pltpu.make_async_copy(val_hbm.at[pl.ds(off, CHUNK)], valb[m], vls[m]).wait()

    def start_gather(m):
        pltpu.async_copy(h_hbm.at[srcb[m]], rows[m], gs[m])

    def wait_gather(m):
        pltpu.make_async_copy(h_hbm.at[srcb[m]], rows[m], gs[m]).wait()

    def scatter_sync(m):
        pltpu.sync_copy(rows[m], acc_shared.at[dstb[m]], add=True)

    def scale(m):
        buf = rows[m]
        vref = valb[m]

        def group_body(g, carry):
            gbase = g * 16
            vv = vref[pl.ds(gbase, 16)]
            for e in range(16):
                v = vv[e]
                for j in range(D // 16):
                    sl = pl.ds(j * 16, 16)
                    buf[gbase + e, sl] = buf[gbase + e, sl] * v
            return carry

        lax.fori_loop(0, CHUNK // 16, group_body, 0)

    start_idx(0, 0)
    start_idx(1, 1)
    wait_idx(0, 0)
    start_gather(0)

    def pair_body(p, carry):
        c0 = 2 * p
        c1 = c0 + 1
        wait_idx(c1, 1)
        start_gather(1)
        wait_gather(0)
        scale(0)
        scatter_sync(0)
        start_idx(c0 + 2, 0)
        wait_gather(1)
        scale(1)
        scatter_sync(1)
        wait_idx(c0 + 2, 0)
        start_gather(0)
        start_idx(jnp.minimum(c1 + 2, NCHUNK - 1), 1)
        return carry

    lax.fori_loop(0, (NCHUNK - 1) // 2, pair_body, 0)

    wait_gather(0)
    scale(0)
    scatter_sync(0)
    wait_idx(NCHUNK - 1, 1)

    plsc.subcore_barrier()

    pltpu.sync_copy(
        acc_shared.at[pl.ds(sid * ROWS_PER_TILE, ROWS_PER_TILE)],
        out_hbm.at[cid, pl.ds(sid * ROWS_PER_TILE, ROWS_PER_TILE)],
    )


@jax.jit
def _spmm(h, src1, dst1, val1):
    mesh = plsc.VectorSubcoreMesh(core_axis_name="c", subcore_axis_name="s")
    f = pl.kernel(
        _spmm_body,
        out_type=jax.ShapeDtypeStruct((NC, NP, D), jnp.float32),
        mesh=mesh,
        scratch_types=[
            [pltpu.VMEM((CHUNK,), jnp.int32) for _ in range(2)],
            [pltpu.VMEM((CHUNK,), jnp.int32) for _ in range(2)],
            [pltpu.VMEM((CHUNK,), jnp.float32) for _ in range(2)],
            [pltpu.VMEM((CHUNK, D), jnp.float32) for _ in range(2)],
            pltpu.VMEM((128, D), jnp.float32),
            pltpu.VMEM_SHARED((NP, D), jnp.float32),
            [pltpu.SemaphoreType.DMA for _ in range(2)],
            [pltpu.SemaphoreType.DMA for _ in range(2)],
            [pltpu.SemaphoreType.DMA for _ in range(2)],
            [pltpu.SemaphoreType.DMA for _ in range(2)],
        ],
    )
    return f(h, src1, dst1, val1)


def kernel(x, adj_indices, adj_values, W, b):
    h = pl.pallas_call(
        _matmul_kernel,
        grid=(10,),
        in_specs=[
            pl.BlockSpec((N // 10, D), lambda i: (i, 0)),
            pl.BlockSpec((D, D), lambda i: (0, 0)),
            pl.BlockSpec((1, D), lambda i: (0, 0)),
        ],
        out_specs=pl.BlockSpec((N // 10, D), lambda i: (i, 0)),
        out_shape=jax.ShapeDtypeStruct((N, D), jnp.float32),
    )(x, W, b.reshape(1, D))

    dst1 = adj_indices[0]
    src1 = adj_indices[1]
    partials = _spmm(h, src1, dst1, adj_values)

    out = pl.pallas_call(
        _add_kernel,
        grid=(10,),
        in_specs=[
            pl.BlockSpec((N // 10, D), lambda i: (i, 0)),
            pl.BlockSpec((N // 10, D), lambda i: (i, 0)),
        ],
        out_specs=pl.BlockSpec((N // 10, D), lambda i: (i, 0)),
        out_shape=jax.ShapeDtypeStruct((N, D), jnp.float32),
    )(partials[0], partials[1])
    return out


# speedup vs baseline: 11.2761x; 1.3541x over previous
import jax
import jax.numpy as jnp
from jax import lax
from jax.experimental import pallas as pl
from jax.experimental.pallas import tpu as pltpu
from jax.experimental.pallas import tpu_sc as plsc

N = 10000
E = 320000
D = 128

NC = 2
NS = 16
NW = NC * NS
EPW = E // NW
CHUNK = 80
NCHUNK = EPW // CHUNK
NP = 10240
ROWS_PER_TILE = NP // NS

NRB = 4
NIB = 8


def _matmul_kernel(x_ref, w_ref, b_ref, o_ref):
    o_ref[...] = (
        jnp.dot(x_ref[...], w_ref[...], preferred_element_type=jnp.float32)
        + b_ref[...]
    )


def _add_kernel(a_ref, b_ref, o_ref):
    o_ref[...] = a_ref[...] + b_ref[...]


def _spmm_body(h_hbm, src_hbm, dst_hbm, val_hbm, out_hbm,
               srcb, dstb, valb, rows, zbuf, acc_shared, srs, dss, vls, gs, ss):
    cid = lax.axis_index("c")
    sid = lax.axis_index("s")
    wid = sid * NC + cid

    zv = jnp.zeros((16,), jnp.float32)

    def zrow(r, carry):
        for j in range(D // 16):
            zbuf[r, pl.ds(j * 16, 16)] = zv
        return carry

    lax.fori_loop(0, 32, zrow, 0)
    for t in range(ROWS_PER_TILE // 32):
        pltpu.sync_copy(
            zbuf,
            acc_shared.at[pl.ds(sid * ROWS_PER_TILE + t * 32, 32)],
        )
    plsc.subcore_barrier()

    ebase = wid * EPW
    LAST = NCHUNK - 1

    def start_sv(c, m):
        off = ebase + c * CHUNK
        pltpu.async_copy(src_hbm.at[pl.ds(off, CHUNK)], srcb[m], srs[m])
        pltpu.async_copy(val_hbm.at[pl.ds(off, CHUNK)], valb[m], vls[m])

    def wait_sv(c, m):
        off = ebase + c * CHUNK
        pltpu.make_async_copy(src_hbm.at[pl.ds(off, CHUNK)], srcb[m], srs[m]).wait()
        pltpu.make_async_copy(val_hbm.at[pl.ds(off, CHUNK)], valb[m], vls[m]).wait()

    def start_dst(c, m):
        off = ebase + c * CHUNK
        pltpu.async_copy(dst_hbm.at[pl.ds(off, CHUNK)], dstb[m], dss[m])

    def wait_dst(c, m):
        off = ebase + c * CHUNK
        pltpu.make_async_copy(dst_hbm.at[pl.ds(off, CHUNK)], dstb[m], dss[m]).wait()

    def start_gather(msrc, mdst, sem):
        pltpu.async_copy(h_hbm.at[srcb[msrc]], rows[mdst], sem)

    def wait_gather(m):
        pltpu.make_async_copy(h_hbm.at[srcb[m]], rows[m], gs[m]).wait()

    def start_scatter(m):
        pltpu.async_copy(rows[m], acc_shared.at[dstb[m]], ss[m], add=True)

    def wait_scatter(m):
        pltpu.make_async_copy(h_hbm.at[pl.ds(0, CHUNK)], rows[m], ss[m]).wait()

    def scale(m):
        buf = rows[m]
        vref = valb[m]

        def group_body(g, carry):
            gbase = g * 16
            vv = vref[pl.ds(gbase, 16)]
            for e in range(16):
                v = vv[e]
                for j in range(D // 16):
                    sl = pl.ds(j * 16, 16)
                    buf[gbase + e, sl] = buf[gbase + e, sl] * v
            return carry

        lax.fori_loop(0, CHUNK // 16, group_body, 0)

    def step(c, k):
        kn = (k + 2) % 4
        cg = jnp.minimum(c + 2, LAST)
        ci = jnp.minimum(c + 4, LAST)
        wait_gather(k)
        scale(k)
        wait_dst(c, k)
        start_scatter(k)
        wait_scatter(kn)
        wait_sv(cg, kn)
        start_gather(kn, kn, gs[kn])
        start_sv(ci, k)
        start_dst(cg, kn)

    for c in range(4):
        start_sv(c, c)
    start_dst(0, 0)
    start_dst(1, 1)
    wait_sv(0, 0)
    start_gather(0, 0, gs[0])
    start_gather(0, 2, ss[2])
    wait_sv(1, 1)
    start_gather(1, 1, gs[1])
    start_gather(1, 3, ss[3])

    def quad_body(p, carry):
        base = 4 * p
        for k in range(4):
            step(base + k, k)
        return carry

    lax.fori_loop(0, NCHUNK // 4, quad_body, 0)
    step(LAST, 0)

    wait_gather(1)
    wait_gather(2)
    wait_sv(LAST, 3)
    wait_sv(LAST, 0)
    wait_dst(LAST, 1)
    wait_dst(LAST, 2)
    wait_scatter(3)
    wait_scatter(0)

    plsc.subcore_barrier()

    pltpu.sync_copy(
        acc_shared.at[pl.ds(sid * ROWS_PER_TILE, ROWS_PER_TILE)],
        out_hbm.at[cid, pl.ds(sid * ROWS_PER_TILE, ROWS_PER_TILE)],
    )


@jax.jit
def _spmm(h, src1, dst1, val1):
    mesh = plsc.VectorSubcoreMesh(core_axis_name="c", subcore_axis_name="s")
    f = pl.kernel(
        _spmm_body,
        out_type=jax.ShapeDtypeStruct((NC, NP, D), jnp.float32),
        mesh=mesh,
        scratch_types=[
            [pltpu.VMEM((CHUNK,), jnp.int32) for _ in range(4)],
            [pltpu.VMEM((CHUNK,), jnp.int32) for _ in range(4)],
            [pltpu.VMEM((CHUNK,), jnp.float32) for _ in range(4)],
            [pltpu.VMEM((CHUNK, D), jnp.float32) for _ in range(4)],
            pltpu.VMEM((32, D), jnp.float32),
            pltpu.VMEM_SHARED((NP, D), jnp.float32),
            [pltpu.SemaphoreType.DMA for _ in range(4)],
            [pltpu.SemaphoreType.DMA for _ in range(4)],
            [pltpu.SemaphoreType.DMA for _ in range(4)],
            [pltpu.SemaphoreType.DMA for _ in range(4)],
            [pltpu.SemaphoreType.DMA for _ in range(4)],
        ],
    )
    return f(h, src1, dst1, val1)


def kernel(x, adj_indices, adj_values, W, b):
    h = pl.pallas_call(
        _matmul_kernel,
        grid=(10,),
        in_specs=[
            pl.BlockSpec((N // 10, D), lambda i: (i, 0)),
            pl.BlockSpec((D, D), lambda i: (0, 0)),
            pl.BlockSpec((1, D), lambda i: (0, 0)),
        ],
        out_specs=pl.BlockSpec((N // 10, D), lambda i: (i, 0)),
        out_shape=jax.ShapeDtypeStruct((N, D), jnp.float32),
    )(x, W, b.reshape(1, D))

    dst1 = adj_indices[0]
    src1 = adj_indices[1]
    partials = _spmm(h, src1, dst1, adj_values)

    out = pl.pallas_call(
        _add_kernel,
        grid=(10,),
        in_specs=[
            pl.BlockSpec((N // 10, D), lambda i: (i, 0)),
            pl.BlockSpec((N // 10, D), lambda i: (i, 0)),
        ],
        out_specs=pl.BlockSpec((N // 10, D), lambda i: (i, 0)),
        out_shape=jax.ShapeDtypeStruct((N, D), jnp.float32),
    )(partials[0], partials[1])
    return out


# speedup vs baseline: 11.4334x; 1.0139x over previous
import jax
import jax.numpy as jnp
from jax import lax
from jax.experimental import pallas as pl
from jax.experimental.pallas import tpu as pltpu
from jax.experimental.pallas import tpu_sc as plsc

N = 10000
E = 320000
D = 128

NC = 2
NS = 16
NW = NC * NS
EPW = E // NW
CHUNK = 80
NCHUNK = EPW // CHUNK
NP = 10240
ROWS_PER_TILE = NP // NS

NRB = 4
NIB = 8


def _matmul_kernel(x_ref, w_ref, b_ref, o_ref):
    o_ref[...] = (
        jnp.dot(x_ref[...], w_ref[...], preferred_element_type=jnp.float32)
        + b_ref[...]
    )


def _add_kernel(a_ref, b_ref, o_ref):
    o_ref[...] = a_ref[...] + b_ref[...]


def _spmm_body(h_hbm, src_hbm, dst_hbm, val_hbm, out_hbm,
               srcb, dstb, valb, rows, zbuf, acc_shared, srs, dss, vls, gs, ss):
    cid = lax.axis_index("c")
    sid = lax.axis_index("s")
    wid = sid * NC + cid

    ebase = wid * EPW
    LAST = NCHUNK - 1

    def start_sv(c, m):
        off = ebase + c * CHUNK
        pltpu.async_copy(src_hbm.at[pl.ds(off, CHUNK)], srcb[m], srs[m])
        pltpu.async_copy(val_hbm.at[pl.ds(off, CHUNK)], valb[m], vls[m])

    def wait_sv(c, m):
        off = ebase + c * CHUNK
        pltpu.make_async_copy(src_hbm.at[pl.ds(off, CHUNK)], srcb[m], srs[m]).wait()
        pltpu.make_async_copy(val_hbm.at[pl.ds(off, CHUNK)], valb[m], vls[m]).wait()

    def start_dst(c, m):
        off = ebase + c * CHUNK
        pltpu.async_copy(dst_hbm.at[pl.ds(off, CHUNK)], dstb[m], dss[m])

    def wait_dst(c, m):
        off = ebase + c * CHUNK
        pltpu.make_async_copy(dst_hbm.at[pl.ds(off, CHUNK)], dstb[m], dss[m]).wait()

    def start_gather(msrc, mdst, sem):
        pltpu.async_copy(h_hbm.at[srcb[msrc]], rows[mdst], sem)

    def wait_gather(m):
        pltpu.make_async_copy(h_hbm.at[srcb[m]], rows[m], gs[m]).wait()

    def start_scatter(m):
        pltpu.async_copy(rows[m], acc_shared.at[dstb[m]], ss[m], add=True)

    def wait_scatter(m):
        pltpu.make_async_copy(h_hbm.at[pl.ds(0, CHUNK)], rows[m], ss[m]).wait()

    def scale(m):
        buf = rows[m]
        vref = valb[m]

        def group_body(g, carry):
            gbase = g * 16
            vv = vref[pl.ds(gbase, 16)]
            for e in range(16):
                v = vv[e]
                for j in range(D // 16):
                    sl = pl.ds(j * 16, 16)
                    buf[gbase + e, sl] = buf[gbase + e, sl] * v
            return carry

        lax.fori_loop(0, CHUNK // 16, group_body, 0)

    def step(c, k):
        kn = (k + 2) % 4
        cg = jnp.minimum(c + 2, LAST)
        ci = jnp.minimum(c + 4, LAST)
        wait_gather(k)
        scale(k)
        wait_dst(c, k)
        start_scatter(k)
        wait_scatter(kn)
        wait_sv(cg, kn)
        start_gather(kn, kn, gs[kn])
        start_sv(ci, k)
        start_dst(cg, kn)

    for c in range(4):
        start_sv(c, c)
    start_dst(0, 0)
    start_dst(1, 1)
    wait_sv(0, 0)
    start_gather(0, 0, gs[0])
    start_gather(0, 2, ss[2])
    wait_sv(1, 1)
    start_gather(1, 1, gs[1])
    start_gather(1, 3, ss[3])

    zv = jnp.zeros((16,), jnp.float32)

    def zrow(r, carry):
        for j in range(D // 16):
            zbuf[r, pl.ds(j * 16, 16)] = zv
        return carry

    lax.fori_loop(0, 32, zrow, 0)
    for t in range(ROWS_PER_TILE // 32):
        pltpu.sync_copy(
            zbuf,
            acc_shared.at[pl.ds(sid * ROWS_PER_TILE + t * 32, 32)],
        )
    plsc.subcore_barrier()

    def quad_body(p, carry):
        base = 4 * p
        for k in range(4):
            step(base + k, k)
        return carry

    lax.fori_loop(0, NCHUNK // 4, quad_body, 0)
    step(LAST, 0)

    wait_gather(1)
    wait_gather(2)
    wait_sv(LAST, 3)
    wait_sv(LAST, 0)
    wait_dst(LAST, 1)
    wait_dst(LAST, 2)
    wait_scatter(3)
    wait_scatter(0)

    plsc.subcore_barrier()

    pltpu.sync_copy(
        acc_shared.at[pl.ds(sid * ROWS_PER_TILE, ROWS_PER_TILE)],
        out_hbm.at[cid, pl.ds(sid * ROWS_PER_TILE, ROWS_PER_TILE)],
    )


@jax.jit
def _spmm(h, src1, dst1, val1):
    mesh = plsc.VectorSubcoreMesh(core_axis_name="c", subcore_axis_name="s")
    f = pl.kernel(
        _spmm_body,
        out_type=jax.ShapeDtypeStruct((NC, NP, D), jnp.float32),
        mesh=mesh,
        scratch_types=[
            [pltpu.VMEM((CHUNK,), jnp.int32) for _ in range(4)],
            [pltpu.VMEM((CHUNK,), jnp.int32) for _ in range(4)],
            [pltpu.VMEM((CHUNK,), jnp.float32) for _ in range(4)],
            [pltpu.VMEM((CHUNK, D), jnp.float32) for _ in range(4)],
            pltpu.VMEM((32, D), jnp.float32),
            pltpu.VMEM_SHARED((NP, D), jnp.float32),
            [pltpu.SemaphoreType.DMA for _ in range(4)],
            [pltpu.SemaphoreType.DMA for _ in range(4)],
            [pltpu.SemaphoreType.DMA for _ in range(4)],
            [pltpu.SemaphoreType.DMA for _ in range(4)],
            [pltpu.SemaphoreType.DMA for _ in range(4)],
        ],
    )
    return f(h, src1, dst1, val1)


def kernel(x, adj_indices, adj_values, W, b):
    h = pl.pallas_call(
        _matmul_kernel,
        grid=(10,),
        in_specs=[
            pl.BlockSpec((N // 10, D), lambda i: (i, 0)),
            pl.BlockSpec((D, D), lambda i: (0, 0)),
            pl.BlockSpec((1, D), lambda i: (0, 0)),
        ],
        out_specs=pl.BlockSpec((N // 10, D), lambda i: (i, 0)),
        out_shape=jax.ShapeDtypeStruct((N, D), jnp.float32),
    )(x, W, b.reshape(1, D))

    dst1 = adj_indices[0]
    src1 = adj_indices[1]
    partials = _spmm(h, src1, dst1, adj_values)

    out = pl.pallas_call(
        _add_kernel,
        grid=(10,),
        in_specs=[
            pl.BlockSpec((N // 10, D), lambda i: (i, 0)),
            pl.BlockSpec((N // 10, D), lambda i: (i, 0)),
        ],
        out_specs=pl.BlockSpec((N // 10, D), lambda i: (i, 0)),
        out_shape=jax.ShapeDtypeStruct((N, D), jnp.float32),
    )(partials[0], partials[1])
    return out


# speedup vs baseline: 11.6653x; 1.0203x over previous
import jax
import jax.numpy as jnp
from jax import lax
from jax.experimental import pallas as pl
from jax.experimental.pallas import tpu as pltpu
from jax.experimental.pallas import tpu_sc as plsc

N = 10000
E = 320000
D = 128

NC = 2
NS = 16
NW = NC * NS
EPW = E // NW
CHUNK = 80
NCHUNK = EPW // CHUNK
NP = 10240
ROWS_PER_TILE = NP // NS

NRB = 4
NIB = 8


def _matmul_kernel(x_ref, w_ref, b_ref, o_ref):
    o_ref[...] = (
        jnp.dot(x_ref[...], w_ref[...], preferred_element_type=jnp.float32)
        + b_ref[...]
    )


def _add_kernel(a_ref, b_ref, o_ref):
    o_ref[...] = a_ref[...] + b_ref[...]


def _spmm_body(h_hbm, src_hbm, dst_hbm, val_hbm, out_hbm,
               srcb, dstb, valb, rows, zbuf, acc_shared, srs, dss, vls, gs, ss):
    cid = lax.axis_index("c")
    sid = lax.axis_index("s")
    wid = sid * NC + cid

    ebase = wid * EPW
    LAST = NCHUNK - 1

    def start_sv(c, m):
        off = ebase + c * CHUNK
        pltpu.async_copy(src_hbm.at[pl.ds(off, CHUNK)], srcb[m], srs[m])
        pltpu.async_copy(val_hbm.at[pl.ds(off, CHUNK)], valb[m], vls[m])

    def wait_sv(c, m):
        off = ebase + c * CHUNK
        pltpu.make_async_copy(src_hbm.at[pl.ds(off, CHUNK)], srcb[m], srs[m]).wait()
        pltpu.make_async_copy(val_hbm.at[pl.ds(off, CHUNK)], valb[m], vls[m]).wait()

    def start_dst(c, m):
        off = ebase + c * CHUNK
        pltpu.async_copy(dst_hbm.at[pl.ds(off, CHUNK)], dstb[m], dss[m])

    def wait_dst(c, m):
        off = ebase + c * CHUNK
        pltpu.make_async_copy(dst_hbm.at[pl.ds(off, CHUNK)], dstb[m], dss[m]).wait()

    def start_gather(msrc, mdst, sem):
        pltpu.async_copy(h_hbm.at[srcb[msrc]], rows[mdst], sem)

    def wait_gather(m):
        pltpu.make_async_copy(h_hbm.at[srcb[m]], rows[m], gs[m]).wait()

    def start_scatter(m):
        pltpu.async_copy(rows[m], acc_shared.at[dstb[m]], ss[m], add=True)

    def wait_scatter(m):
        pltpu.make_async_copy(h_hbm.at[pl.ds(0, CHUNK)], rows[m], ss[m]).wait()

    def scale(m):
        buf = rows[m]
        vref = valb[m]

        def group_body(g, carry):
            gbase = g * 16
            vv = vref[pl.ds(gbase, 16)]
            for e in range(16):
                v = vv[e]
                for j in range(D // 16):
                    sl = pl.ds(j * 16, 16)
                    buf[gbase + e, sl] = buf[gbase + e, sl] * v
            return carry

        lax.fori_loop(0, CHUNK // 16, group_body, 0)

    def step(c, k):
        kn = (k + 2) % 4
        cg = jnp.minimum(c + 2, LAST)
        ci = jnp.minimum(c + 4, LAST)
        wait_gather(k)
        wait_dst(c, k)
        wait_scatter(kn)
        wait_sv(cg, kn)
        start_gather(kn, kn, gs[kn])
        start_dst(cg, kn)
        scale(k)
        start_scatter(k)
        start_sv(ci, k)

    for c in range(4):
        start_sv(c, c)
    start_dst(0, 0)
    start_dst(1, 1)
    wait_sv(0, 0)
    start_gather(0, 0, gs[0])
    start_gather(0, 2, ss[2])
    wait_sv(1, 1)
    start_gather(1, 1, gs[1])
    start_gather(1, 3, ss[3])

    zv = jnp.zeros((16,), jnp.float32)

    def zrow(r, carry):
        for j in range(D // 16):
            zbuf[r, pl.ds(j * 16, 16)] = zv
        return carry

    lax.fori_loop(0, 32, zrow, 0)
    for t in range(ROWS_PER_TILE // 32):
        pltpu.sync_copy(
            zbuf,
            acc_shared.at[pl.ds(sid * ROWS_PER_TILE + t * 32, 32)],
        )
    plsc.subcore_barrier()

    def quad_body(p, carry):
        base = 4 * p
        for k in range(4):
            step(base + k, k)
        return carry

    lax.fori_loop(0, NCHUNK // 4, quad_body, 0)
    step(LAST, 0)

    wait_gather(1)
    wait_gather(2)
    wait_sv(LAST, 3)
    wait_sv(LAST, 0)
    wait_dst(LAST, 1)
    wait_dst(LAST, 2)
    wait_scatter(3)
    wait_scatter(0)

    plsc.subcore_barrier()

    pltpu.sync_copy(
        acc_shared.at[pl.ds(sid * ROWS_PER_TILE, ROWS_PER_TILE)],
        out_hbm.at[cid, pl.ds(sid * ROWS_PER_TILE, ROWS_PER_TILE)],
    )


@jax.jit
def _spmm(h, src1, dst1, val1):
    mesh = plsc.VectorSubcoreMesh(core_axis_name="c", subcore_axis_name="s")
    f = pl.kernel(
        _spmm_body,
        out_type=jax.ShapeDtypeStruct((NC, NP, D), jnp.float32),
        mesh=mesh,
        scratch_types=[
            [pltpu.VMEM((CHUNK,), jnp.int32) for _ in range(4)],
            [pltpu.VMEM((CHUNK,), jnp.int32) for _ in range(4)],
            [pltpu.VMEM((CHUNK,), jnp.float32) for _ in range(4)],
            [pltpu.VMEM((CHUNK, D), jnp.float32) for _ in range(4)],
            pltpu.VMEM((32, D), jnp.float32),
            pltpu.VMEM_SHARED((NP, D), jnp.float32),
            [pltpu.SemaphoreType.DMA for _ in range(4)],
            [pltpu.SemaphoreType.DMA for _ in range(4)],
            [pltpu.SemaphoreType.DMA for _ in range(4)],
            [pltpu.SemaphoreType.DMA for _ in range(4)],
            [pltpu.SemaphoreType.DMA for _ in range(4)],
        ],
    )
    return f(h, src1, dst1, val1)


def kernel(x, adj_indices, adj_values, W, b):
    h = pl.pallas_call(
        _matmul_kernel,
        grid=(10,),
        in_specs=[
            pl.BlockSpec((N // 10, D), lambda i: (i, 0)),
            pl.BlockSpec((D, D), lambda i: (0, 0)),
            pl.BlockSpec((1, D), lambda i: (0, 0)),
        ],
        out_specs=pl.BlockSpec((N // 10, D), lambda i: (i, 0)),
        out_shape=jax.ShapeDtypeStruct((N, D), jnp.float32),
    )(x, W, b.reshape(1, D))

    dst1 = adj_indices[0]
    src1 = adj_indices[1]
    partials = _spmm(h, src1, dst1, adj_values)

    out = pl.pallas_call(
        _add_kernel,
        grid=(10,),
        in_specs=[
            pl.BlockSpec((N // 10, D), lambda i: (i, 0)),
            pl.BlockSpec((N // 10, D), lambda i: (i, 0)),
        ],
        out_specs=pl.BlockSpec((N // 10, D), lambda i: (i, 0)),
        out_shape=jax.ShapeDtypeStruct((N, D), jnp.float32),
    )(partials[0], partials[1])
    return out
